# baseline (speedup 1.0000x reference)
import functools

import jax
import jax.numpy as jnp
from jax import lax
from jax.experimental import pallas as pl
from jax.experimental.pallas import tpu as pltpu
from jax.experimental.pallas import tpu_sc as plsc

_B = 16384
_L = 200
_EMB = 64
_NLAB = 50
_VROWS = 100001
_W32 = _EMB // 2

_NC = 2
_NS = 16
_NW = _NC * _NS
_SPW = _B // _NW
_HALF = _L // 2
_CH = 32
_NCHUNK = _SPW // _CH

_mesh = plsc.VectorSubcoreMesh(
    core_axis_name="c", subcore_axis_name="s",
    num_cores=_NC, num_subcores=_NS)


_PSTRIP = 6400
_PCHUNK = 128
_NPCH = _PSTRIP // _PCHUNK


@functools.partial(
    pl.kernel,
    out_type=(jax.ShapeDtypeStruct((_B, _EMB), jnp.float32),
              jax.ShapeDtypeStruct((_VROWS, _W32), jnp.int32)),
    mesh=_mesh,
    scratch_types=[
        pltpu.VMEM((2, _CH, 2, _HALF), jnp.int32),
        pltpu.VMEM((8, _L, _W32), jnp.int32),
        pltpu.VMEM((_SPW, _EMB), jnp.float32),
        pltpu.VMEM((_EMB,), jnp.float32),
        pltpu.VMEM((2, _PCHUNK, _EMB), jnp.float32),
        pltpu.VMEM((2, _PCHUNK, _W32), jnp.int32),
        pltpu.SemaphoreType.DMA,
        pltpu.SemaphoreType.DMA,
        pltpu.SemaphoreType.DMA,
    ],
    compiler_params=pltpu.CompilerParams(use_tc_tiling_on_sc=False,
                                         needs_layout_passes=False),
)
def _sc_pool(x_hbm, emb_hbm, bias_hbm, pooled_hbm, table_hbm, idx_v, rows_v,
             out_v, bias_v, pk_in, pk_out, sem_g, sem_i, sem_p):
    wid = lax.axis_index("s") * _NC + lax.axis_index("c")
    base = wid * _SPW

    strip0 = jnp.minimum(lax.axis_index("s") * _PSTRIP, _VROWS - _PSTRIP)

    def pack_rows(c, carry):
        buf = c % 2

        @pl.when(c + 1 < _NPCH)
        def _():
            pltpu.async_copy(
                emb_hbm.at[pl.ds(strip0 + (c + 1) * _PCHUNK, _PCHUNK)],
                pk_in.at[(c + 1) % 2], sem_p)

        pltpu.make_async_copy(emb_hbm.at[pl.ds(0, _PCHUNK)],
                              pk_in.at[buf], sem_p).wait()

        def row_body(r, carry2):
            def rnd(v):
                u = plsc.bitcast(v, jnp.int32)
                t = lax.shift_right_logical(u, 16) & 1
                return lax.shift_right_logical(u + 32767 + t, 16)

            v0 = rnd(pk_in[buf, r, pl.ds(0, 16)])
            v1 = rnd(pk_in[buf, r, pl.ds(16, 16)])
            v2 = rnd(pk_in[buf, r, pl.ds(32, 16)])
            v3 = rnd(pk_in[buf, r, pl.ds(48, 16)])
            pk_out[buf, r, pl.ds(0, 16)] = v0 | (v1 << 16)
            pk_out[buf, r, pl.ds(16, 16)] = v2 | (v3 << 16)
            return carry2

        lax.fori_loop(0, _PCHUNK, row_body, 0)
        pltpu.sync_copy(pk_out.at[buf],
                        table_hbm.at[pl.ds(strip0 + c * _PCHUNK, _PCHUNK)])
        return carry

    pltpu.async_copy(emb_hbm.at[pl.ds(strip0, _PCHUNK)], pk_in.at[0], sem_p)
    lax.fori_loop(0, _NPCH, pack_rows, 0)
    plsc.subcore_barrier()


    def issue_sample(cbuf, s_local, rbuf):
        pltpu.async_copy(table_hbm.at[idx_v.at[cbuf, s_local, 0]],
                         rows_v.at[rbuf, pl.ds(0, _HALF)], sem_g)
        pltpu.async_copy(table_hbm.at[idx_v.at[cbuf, s_local, 1]],
                         rows_v.at[rbuf, pl.ds(_HALF, _HALF)], sem_g)

    def wait_sample(rbuf):
        pltpu.make_async_copy(table_hbm.at[pl.ds(0, _L)],
                              rows_v.at[rbuf], sem_g).wait()

    hi_mask = jnp.int32(-65536)

    def accumulate(rbuf, out_row):
        def acc_body(r, accs):
            accs = list(accs)
            for j in range(4):
                row = 4 * r + j
                v0 = rows_v[rbuf, row, pl.ds(0, 16)]
                v1 = rows_v[rbuf, row, pl.ds(16, 16)]
                a0, a1, a2, a3 = accs[4 * j:4 * j + 4]
                accs[4 * j + 0] = a0 + plsc.bitcast(v0 << 16, jnp.float32)
                accs[4 * j + 1] = a1 + plsc.bitcast(v0 & hi_mask,
                                                    jnp.float32)
                accs[4 * j + 2] = a2 + plsc.bitcast(v1 << 16, jnp.float32)
                accs[4 * j + 3] = a3 + plsc.bitcast(v1 & hi_mask,
                                                    jnp.float32)
            return tuple(accs)

        z = jnp.zeros((16,), jnp.float32)
        accs = lax.fori_loop(0, _L // 4, acc_body, (z,) * 16)
        out_v[out_row, pl.ds(0, 16)] = (
            accs[0] + accs[4] + accs[8] + accs[12] + bias_v[pl.ds(0, 16)])
        out_v[out_row, pl.ds(16, 16)] = (
            accs[1] + accs[5] + accs[9] + accs[13] + bias_v[pl.ds(16, 16)])
        out_v[out_row, pl.ds(32, 16)] = (
            accs[2] + accs[6] + accs[10] + accs[14] + bias_v[pl.ds(32, 16)])
        out_v[out_row, pl.ds(48, 16)] = (
            accs[3] + accs[7] + accs[11] + accs[15] + bias_v[pl.ds(48, 16)])

    def chunk_pass(c, cbuf):
        nxt = 1 - cbuf

        @pl.when(c + 1 < _NCHUNK)
        def _():
            pltpu.async_copy(x_hbm.at[pl.ds(base + (c + 1) * _CH, _CH)],
                             idx_v.at[nxt], sem_i)

        out0 = c * _CH

        def duo(d, carry):
            for p in (0, 1):
                s0 = 8 * d + 4 * p
                for b in range(4):
                    issue_sample(cbuf, s0 + 4 + b, 4 * (1 - p) + b)
                for b in range(4):
                    wait_sample(4 * p + b)
                    accumulate(4 * p + b, out0 + s0 + b)
            return carry

        lax.fori_loop(0, _CH // 8 - 1, duo, 0)

        for b in range(4):
            issue_sample(cbuf, _CH - 4 + b, 4 + b)
        for b in range(4):
            wait_sample(b)
            accumulate(b, out0 + _CH - 8 + b)

        @pl.when(c + 1 < _NCHUNK)
        def _():
            pltpu.make_async_copy(x_hbm.at[pl.ds(base, _CH)],
                                  idx_v.at[nxt], sem_i).wait()
            for b in range(4):
                issue_sample(nxt, b, b)

        for b in range(4):
            wait_sample(4 + b)
            accumulate(4 + b, out0 + _CH - 4 + b)

    pltpu.sync_copy(bias_hbm, bias_v)
    pltpu.sync_copy(x_hbm.at[pl.ds(base, _CH)], idx_v.at[0])
    for _b in range(4):
        issue_sample(0, _b, _b)

    def outer(c2, carry):
        chunk_pass(2 * c2, 0)
        chunk_pass(2 * c2 + 1, 1)
        return carry

    lax.fori_loop(0, _NCHUNK // 2, outer, 0)

    pltpu.sync_copy(out_v, pooled_hbm.at[pl.ds(base, _SPW)])


def _mm_body(p_ref, w_ref, b_ref, o_ref):
    o_ref[...] = (
        jnp.dot(p_ref[...] * (1.0 / _L), w_ref[...],
                preferred_element_type=jnp.float32)
        + b_ref[...])


_mm = pl.pallas_call(
    _mm_body,
    out_shape=jax.ShapeDtypeStruct((_B, _NLAB), jnp.float32),
    grid=(8,),
    in_specs=[
        pl.BlockSpec((_B // 8, _EMB), lambda i: (i, 0)),
        pl.BlockSpec((_EMB, _NLAB), lambda i: (0, 0)),
        pl.BlockSpec((1, _NLAB), lambda i: (0, 0)),
    ],
    out_specs=pl.BlockSpec((_B // 8, _NLAB), lambda i: (i, 0)),
)


def kernel(x, emb_table, fc_w, fc_b):
    x3 = x.reshape(_B, 2, _HALF)
    zbias = jnp.zeros((_EMB,), jnp.float32)
    pooled, _ = _sc_pool(x3, emb_table, zbias)
    return _mm(pooled, fc_w.T, fc_b.reshape(1, _NLAB))


# speedup vs baseline: 1.1193x; 1.1193x over previous
import functools

import jax
import jax.numpy as jnp
from jax import lax
from jax.experimental import pallas as pl
from jax.experimental.pallas import tpu as pltpu
from jax.experimental.pallas import tpu_sc as plsc

_B = 16384
_L = 200
_EMB = 64
_NLAB = 50
_VROWS = 100001
_W32 = _EMB // 2

_NC = 2
_NS = 16
_NW = _NC * _NS
_SPW = _B // _NW
_HALF = _L // 2
_CH = 32
_NCHUNK = _SPW // _CH

_mesh = plsc.VectorSubcoreMesh(
    core_axis_name="c", subcore_axis_name="s",
    num_cores=_NC, num_subcores=_NS)


@functools.partial(
    pl.kernel,
    out_type=jax.ShapeDtypeStruct((_B, _EMB), jnp.float32),
    mesh=_mesh,
    scratch_types=[
        pltpu.VMEM((2, _CH, 2, _HALF), jnp.int32),
        pltpu.VMEM((8, _L, _W32), jnp.int32),
        pltpu.VMEM((_SPW, _EMB), jnp.float32),
        pltpu.VMEM((_EMB,), jnp.float32),
        pltpu.SemaphoreType.DMA,
        pltpu.SemaphoreType.DMA,
    ],
    compiler_params=pltpu.CompilerParams(use_tc_tiling_on_sc=False,
                                         needs_layout_passes=False),
)
def _sc_pool(x_hbm, table_hbm, bias_hbm, pooled_hbm, idx_v, rows_v, out_v,
             bias_v, sem_g, sem_i):
    wid = lax.axis_index("s") * _NC + lax.axis_index("c")
    base = wid * _SPW

    def issue_sample(cbuf, s_local, rbuf):
        pltpu.async_copy(table_hbm.at[idx_v.at[cbuf, s_local, 0]],
                         rows_v.at[rbuf, pl.ds(0, _HALF)], sem_g)
        pltpu.async_copy(table_hbm.at[idx_v.at[cbuf, s_local, 1]],
                         rows_v.at[rbuf, pl.ds(_HALF, _HALF)], sem_g)

    def wait_sample(rbuf):
        pltpu.make_async_copy(table_hbm.at[pl.ds(0, _L)],
                              rows_v.at[rbuf], sem_g).wait()

    hi_mask = jnp.int32(-65536)

    def accumulate(rbuf, out_row):
        def acc_body(r, accs):
            accs = list(accs)
            for j in range(4):
                row = 4 * r + j
                v0 = rows_v[rbuf, row, pl.ds(0, 16)]
                v1 = rows_v[rbuf, row, pl.ds(16, 16)]
                a0, a1, a2, a3 = accs[4 * j:4 * j + 4]
                accs[4 * j + 0] = a0 + plsc.bitcast(v0 << 16, jnp.float32)
                accs[4 * j + 1] = a1 + plsc.bitcast(v0 & hi_mask,
                                                    jnp.float32)
                accs[4 * j + 2] = a2 + plsc.bitcast(v1 << 16, jnp.float32)
                accs[4 * j + 3] = a3 + plsc.bitcast(v1 & hi_mask,
                                                    jnp.float32)
            return tuple(accs)

        z = jnp.zeros((16,), jnp.float32)
        accs = lax.fori_loop(0, _L // 4, acc_body, (z,) * 16)
        out_v[out_row, pl.ds(0, 16)] = (
            accs[0] + accs[4] + accs[8] + accs[12] + bias_v[pl.ds(0, 16)])
        out_v[out_row, pl.ds(16, 16)] = (
            accs[1] + accs[5] + accs[9] + accs[13] + bias_v[pl.ds(16, 16)])
        out_v[out_row, pl.ds(32, 16)] = (
            accs[2] + accs[6] + accs[10] + accs[14] + bias_v[pl.ds(32, 16)])
        out_v[out_row, pl.ds(48, 16)] = (
            accs[3] + accs[7] + accs[11] + accs[15] + bias_v[pl.ds(48, 16)])

    def chunk_pass(c, cbuf):
        nxt = 1 - cbuf

        @pl.when(c + 1 < _NCHUNK)
        def _():
            pltpu.async_copy(x_hbm.at[pl.ds(base + (c + 1) * _CH, _CH)],
                             idx_v.at[nxt], sem_i)

        out0 = c * _CH

        def duo(d, carry):
            for p in (0, 1):
                s0 = 8 * d + 4 * p
                for b in range(4):
                    issue_sample(cbuf, s0 + 4 + b, 4 * (1 - p) + b)
                for b in range(4):
                    wait_sample(4 * p + b)
                    accumulate(4 * p + b, out0 + s0 + b)
            return carry

        lax.fori_loop(0, _CH // 8 - 1, duo, 0)

        for b in range(4):
            issue_sample(cbuf, _CH - 4 + b, 4 + b)
        for b in range(4):
            wait_sample(b)
            accumulate(b, out0 + _CH - 8 + b)

        @pl.when(c + 1 < _NCHUNK)
        def _():
            pltpu.make_async_copy(x_hbm.at[pl.ds(base, _CH)],
                                  idx_v.at[nxt], sem_i).wait()
            for b in range(4):
                issue_sample(nxt, b, b)

        for b in range(4):
            wait_sample(4 + b)
            accumulate(4 + b, out0 + _CH - 4 + b)

    pltpu.sync_copy(bias_hbm, bias_v)
    pltpu.sync_copy(x_hbm.at[pl.ds(base, _CH)], idx_v.at[0])
    for _b in range(4):
        issue_sample(0, _b, _b)

    def outer(c2, carry):
        chunk_pass(2 * c2, 0)
        chunk_pass(2 * c2 + 1, 1)
        return carry

    lax.fori_loop(0, _NCHUNK // 2, outer, 0)

    pltpu.sync_copy(out_v, pooled_hbm.at[pl.ds(base, _SPW)])


def _pack_table(emb_table):
    tb = emb_table.astype(jnp.bfloat16)
    h0 = jnp.stack([tb[:, 0:16], tb[:, 16:32]], axis=-1).reshape(_VROWS, 32)
    h1 = jnp.stack([tb[:, 32:48], tb[:, 48:64]], axis=-1).reshape(_VROWS, 32)
    packed = jnp.concatenate([h0, h1], axis=1).reshape(_VROWS, _W32, 2)
    return lax.bitcast_convert_type(packed, jnp.int32)


def _mm_body(p_ref, w_ref, b_ref, o_ref):
    o_ref[...] = (
        jnp.dot(p_ref[...] * (1.0 / _L), w_ref[...],
                preferred_element_type=jnp.float32)
        + b_ref[...])


_mm = pl.pallas_call(
    _mm_body,
    out_shape=jax.ShapeDtypeStruct((_B, _NLAB), jnp.float32),
    grid=(8,),
    in_specs=[
        pl.BlockSpec((_B // 8, _EMB), lambda i: (i, 0)),
        pl.BlockSpec((_EMB, _NLAB), lambda i: (0, 0)),
        pl.BlockSpec((1, _NLAB), lambda i: (0, 0)),
    ],
    out_specs=pl.BlockSpec((_B // 8, _NLAB), lambda i: (i, 0)),
)


def kernel(x, emb_table, fc_w, fc_b):
    x3 = x.reshape(_B, 2, _HALF)
    zbias = jnp.zeros((_EMB,), jnp.float32)
    pooled = _sc_pool(x3, _pack_table(emb_table), zbias)
    return _mm(pooled, fc_w.T, fc_b.reshape(1, _NLAB))
